# baseline (speedup 1.0000x reference)
import jax
import jax.numpy as jnp
from jax import lax
from jax.experimental import pallas as pl
from jax.experimental.pallas import tpu as pltpu

_BIN_SIZE = 4096
_CHUNK = 512
_NBUF = 8
_AHEAD = 4


def _ffd_bins(lengths, bin_size):
    order = sorted(range(len(lengths)), key=lambda i: lengths[i], reverse=True)
    bins = [[]]
    for idx in order:
        L = lengths[idx]
        placed = False
        for b in bins:
            if sum(lengths[j] for j in b) + L <= bin_size:
                b.append(idx)
                placed = True
                break
        if not placed:
            bins.append([idx])
    return bins


def kernel(seq0, seq1, seq2, seq3, seq4, seq5, seq6, seq7):
    seqs = [seq0, seq1, seq2, seq3, seq4, seq5, seq6, seq7]
    lengths = [int(s.shape[0]) for s in seqs]
    hidden = int(seqs[0].shape[1])
    bins = _ffd_bins(lengths, _BIN_SIZE)
    used = [sum(lengths[j] for j in b) for b in bins]
    max_len = max(used)
    nbins = len(bins)

    copies = []
    pads = []
    for b, members in enumerate(bins):
        off = 0
        for j in members:
            copies.append((b, off, j))
            off += lengths[j]
        if off < max_len:
            pads.append((b, off, max_len - off))

    per_bin = [[] for _ in range(nbins)]
    for (b, dst0, j) in copies:
        for rel in range(0, lengths[j], _CHUNK):
            cnt = min(_CHUNK, lengths[j] - rel)
            per_bin[b].append((j, b, dst0 + rel, rel, cnt))
    plan = []
    while any(per_bin):
        for q in per_bin:
            if q:
                plan.append(q.pop(0))
    nchunks = len(plan)

    nzdma = sum((p + _CHUNK - 1) // _CHUNK for (_, _, p) in pads)

    def body(*refs):
        seq_refs = refs[:8]
        out_ref, mask_ref = refs[8], refs[9]
        bufs = list(refs[10:10 + _NBUF])
        zbuf, mbuf = refs[10 + _NBUF], refs[11 + _NBUF]
        rsems = refs[12 + _NBUF]
        wsems = refs[13 + _NBUF]
        zsems = refs[14 + _NBUF]

        rh = [None] * nchunks
        wh = [None] * nchunks

        def start_read(i):
            j, b, dst, src, cnt = plan[i]
            rh[i] = pltpu.make_async_copy(
                seq_refs[j].at[pl.ds(src, cnt), :],
                bufs[i % _NBUF].at[pl.ds(0, cnt)],
                rsems.at[i % _NBUF],
            )
            rh[i].start()

        def start_write(i):
            j, b, dst, src, cnt = plan[i]
            wh[i] = pltpu.make_async_copy(
                bufs[i % _NBUF].at[pl.ds(0, cnt)],
                out_ref.at[b, pl.ds(dst, cnt), :],
                wsems.at[i % _NBUF],
            )
            wh[i].start()

        for i in range(min(_AHEAD, nchunks)):
            start_read(i)

        zbuf[...] = jnp.zeros((_CHUNK, hidden), jnp.float32)
        col = lax.broadcasted_iota(jnp.int32, (1, max_len), 1)
        for b in range(nbins):
            mbuf[pl.ds(b, 1), :] = jnp.where(
                col < used[b], jnp.float32(1.0), jnp.float32(0.0))

        aux = []
        zi = 0
        for (b, off, p) in pads:
            for z0 in range(0, p, _CHUNK):
                zc = min(_CHUNK, p - z0)
                h = pltpu.make_async_copy(
                    zbuf.at[pl.ds(0, zc)],
                    out_ref.at[b, pl.ds(off + z0, zc), :],
                    zsems.at[zi],
                )
                h.start()
                aux.append(h)
                zi += 1
        h = pltpu.make_async_copy(mbuf, mask_ref, zsems.at[zi])
        h.start()
        aux.append(h)

        waited = set()
        for i in range(nchunks):
            if i + _AHEAD < nchunks:
                prev = i + _AHEAD - _NBUF
                if prev >= 0:
                    wh[prev].wait()
                    waited.add(prev)
                start_read(i + _AHEAD)
            rh[i].wait()
            start_write(i)

        for i in range(nchunks):
            if i not in waited:
                wh[i].wait()
        for h in aux:
            h.wait()

    return pl.pallas_call(
        body,
        in_specs=[pl.BlockSpec(memory_space=pltpu.MemorySpace.HBM)] * 8,
        out_specs=(
            pl.BlockSpec(memory_space=pltpu.MemorySpace.HBM),
            pl.BlockSpec(memory_space=pltpu.MemorySpace.HBM),
        ),
        out_shape=(
            jax.ShapeDtypeStruct((nbins, max_len, hidden), jnp.float32),
            jax.ShapeDtypeStruct((nbins, max_len), jnp.float32),
        ),
        scratch_shapes=(
            [pltpu.VMEM((_CHUNK, hidden), jnp.float32)] * _NBUF
            + [
                pltpu.VMEM((_CHUNK, hidden), jnp.float32),
                pltpu.VMEM((nbins, max_len), jnp.float32),
                pltpu.SemaphoreType.DMA((_NBUF,)),
                pltpu.SemaphoreType.DMA((_NBUF,)),
                pltpu.SemaphoreType.DMA((nzdma + 1,)),
            ]
        ),
    )(*seqs)


# speedup vs baseline: 1.0011x; 1.0011x over previous
import jax
import jax.numpy as jnp
from jax import lax
from jax.experimental import pallas as pl
from jax.experimental.pallas import tpu as pltpu

_BIN_SIZE = 4096
_CHUNK = 512
_NBUF = 8
_AHEAD = 4


def _ffd_bins(lengths, bin_size):
    order = sorted(range(len(lengths)), key=lambda i: lengths[i], reverse=True)
    bins = [[]]
    for idx in order:
        L = lengths[idx]
        placed = False
        for b in bins:
            if sum(lengths[j] for j in b) + L <= bin_size:
                b.append(idx)
                placed = True
                break
        if not placed:
            bins.append([idx])
    return bins


def kernel(seq0, seq1, seq2, seq3, seq4, seq5, seq6, seq7):
    seqs = [seq0, seq1, seq2, seq3, seq4, seq5, seq6, seq7]
    lengths = [int(s.shape[0]) for s in seqs]
    hidden = int(seqs[0].shape[1])
    bins = _ffd_bins(lengths, _BIN_SIZE)
    used = [sum(lengths[j] for j in b) for b in bins]
    max_len = max(used)
    nbins = len(bins)

    copies = []
    pads = []
    for b, members in enumerate(bins):
        off = 0
        for j in members:
            copies.append((b, off, j))
            off += lengths[j]
        if off < max_len:
            pads.append((b, off, max_len - off))

    plan = []
    for (b, dst0, j) in copies:
        for rel in range(0, lengths[j], _CHUNK):
            cnt = min(_CHUNK, lengths[j] - rel)
            plan.append((j, b, dst0 + rel, rel, cnt))
    nchunks = len(plan)

    nzdma = sum((p + _CHUNK - 1) // _CHUNK for (_, _, p) in pads)

    def body(*refs):
        seq_refs = refs[:8]
        out_ref, mask_ref = refs[8], refs[9]
        bufs = list(refs[10:10 + _NBUF])
        zbuf, mbuf = refs[10 + _NBUF], refs[11 + _NBUF]
        rsems = refs[12 + _NBUF]
        wsems = refs[13 + _NBUF]
        zsems = refs[14 + _NBUF]

        rh = [None] * nchunks
        wh = [None] * nchunks

        def start_read(i):
            j, b, dst, src, cnt = plan[i]
            rh[i] = pltpu.make_async_copy(
                seq_refs[j].at[pl.ds(src, cnt), :],
                bufs[i % _NBUF].at[pl.ds(0, cnt)],
                rsems.at[i % _NBUF],
            )
            rh[i].start()

        def start_write(i):
            j, b, dst, src, cnt = plan[i]
            wh[i] = pltpu.make_async_copy(
                bufs[i % _NBUF].at[pl.ds(0, cnt)],
                out_ref.at[b, pl.ds(dst, cnt), :],
                wsems.at[i % _NBUF],
            )
            wh[i].start()

        for i in range(min(_AHEAD, nchunks)):
            start_read(i)

        zbuf[...] = jnp.zeros((_CHUNK, hidden), jnp.float32)
        col = lax.broadcasted_iota(jnp.int32, (1, max_len), 1)
        for b in range(nbins):
            mbuf[pl.ds(b, 1), :] = jnp.where(
                col < used[b], jnp.float32(1.0), jnp.float32(0.0))

        aux = []
        zi = 0
        for (b, off, p) in pads:
            for z0 in range(0, p, _CHUNK):
                zc = min(_CHUNK, p - z0)
                h = pltpu.make_async_copy(
                    zbuf.at[pl.ds(0, zc)],
                    out_ref.at[b, pl.ds(off + z0, zc), :],
                    zsems.at[zi],
                )
                h.start()
                aux.append(h)
                zi += 1
        h = pltpu.make_async_copy(mbuf, mask_ref, zsems.at[zi])
        h.start()
        aux.append(h)

        waited = set()
        for i in range(nchunks):
            if i + _AHEAD < nchunks:
                prev = i + _AHEAD - _NBUF
                if prev >= 0:
                    wh[prev].wait()
                    waited.add(prev)
                start_read(i + _AHEAD)
            rh[i].wait()
            start_write(i)

        for i in range(nchunks):
            if i not in waited:
                wh[i].wait()
        for h in aux:
            h.wait()

    return pl.pallas_call(
        body,
        in_specs=[pl.BlockSpec(memory_space=pltpu.MemorySpace.HBM)] * 8,
        out_specs=(
            pl.BlockSpec(memory_space=pltpu.MemorySpace.HBM),
            pl.BlockSpec(memory_space=pltpu.MemorySpace.HBM),
        ),
        out_shape=(
            jax.ShapeDtypeStruct((nbins, max_len, hidden), jnp.float32),
            jax.ShapeDtypeStruct((nbins, max_len), jnp.float32),
        ),
        scratch_shapes=(
            [pltpu.VMEM((_CHUNK, hidden), jnp.float32)] * _NBUF
            + [
                pltpu.VMEM((_CHUNK, hidden), jnp.float32),
                pltpu.VMEM((nbins, max_len), jnp.float32),
                pltpu.SemaphoreType.DMA((_NBUF,)),
                pltpu.SemaphoreType.DMA((_NBUF,)),
                pltpu.SemaphoreType.DMA((nzdma + 1,)),
            ]
        ),
    )(*seqs)
